# baseline (speedup 1.0000x reference)
import jax
import jax.numpy as jnp
from jax import lax
from jax.experimental import pallas as pl
from jax.experimental.pallas import tpu as pltpu
from jax.experimental.pallas import tpu_sc as plsc

N_TILE = 2048
WIN = 32


def _cross_rows(ax, ay, az, bx, by, bz):
    return (ay * bz - az * by, az * bx - ax * bz, ax * by - ay * bx)


def _solve_from_moments(mom, nn, cell):
    ninv = 1.0 / nn
    ax_, ay_, az_ = mom[0:1], mom[1:2], mom[2:3]
    px_, py_, pz_ = mom[3:4], mom[4:5], mom[5:6]
    cx_, cy_, cz_ = mom[6:7], mom[7:8], mom[8:9]
    q = mom[9:10]
    oxx, oyy, ozz = mom[10:11], mom[11:12], mom[12:13]
    oxy, oxz, oyz = mom[13:14], mom[14:15], mom[15:16]

    mean_x, mean_y, mean_z = ax_ * ninv, ay_ * ninv, az_ * ninv
    com_x, com_y, com_z = px_ * ninv, py_ * ninv, pz_ * ninv
    kx, ky, kz = _cross_rows(px_, py_, pz_, ax_, ay_, az_)
    tx = cx_ - kx * ninv
    ty = cy_ - ky * ninv
    tz = cz_ - kz * ninv
    s = q - (px_ * px_ + py_ * py_ + pz_ * pz_) * ninv
    a = oxx - px_ * px_ * ninv - s
    d = oyy - py_ * py_ * ninv - s
    f = ozz - pz_ * pz_ * ninv - s
    b = oxy - px_ * py_ * ninv
    c = oxz - px_ * pz_ * ninv
    e = oyz - py_ * pz_ * ninv

    det = a * (d * f - e * e) - b * (b * f - e * c) + c * (b * e - d * c)
    dinv = 1.0 / det
    i00 = d * f - e * e
    i01 = c * e - b * f
    i02 = b * e - c * d
    i11 = a * f - c * c
    i12 = b * c - a * e
    i22 = a * d - b * b
    mux = -(i00 * tx + i01 * ty + i02 * tz) * dinv
    muy = -(i01 * tx + i11 * ty + i12 * tz) * dinv
    muz = -(i02 * tx + i12 * ty + i22 * tz) * dinv
    rx = tx + a * mux + b * muy + c * muz
    ry = ty + b * mux + d * muy + e * muz
    rz = tz + c * mux + e * muy + f * muz
    mux = mux - (i00 * rx + i01 * ry + i02 * rz) * dinv
    muy = muy - (i01 * rx + i11 * ry + i12 * rz) * dinv
    muz = muz - (i02 * rx + i12 * ry + i22 * rz) * dinv

    nopbc = jnp.all(cell == 0.0, axis=0, keepdims=True)
    zero = jnp.zeros_like(mux)
    mux = jnp.where(nopbc, mux, zero)
    muy = jnp.where(nopbc, muy, zero)
    muz = jnp.where(nopbc, muz, zero)

    return jnp.concatenate(
        [mean_x, mean_y, mean_z, com_x, com_y, com_z, mux, muy, muz,
         zero, zero, zero, zero, zero, zero, zero], axis=0)


def _mlp_moments_body(x_ref, w1_ref, b1_ref, w2_ref, b2_ref, pos_ref,
                      sw_ref, ew_ref, bj_ref, nn_ref, cell_ref,
                      pred_ref, table_ref, mom_ref):
    t = pl.program_id(0)
    nt = pl.num_programs(0)
    h = jax.nn.gelu(jnp.dot(x_ref[...], w1_ref[...],
                            preferred_element_type=jnp.float32) + b1_ref[...])
    pred = jax.lax.dot_general(w2_ref[...], h, (((0,), (1,)), ((), ())),
                               preferred_element_type=jnp.float32) + b2_ref[...]
    pred_ref[...] = pred

    pos = pos_ref[...]
    px, py, pz = pos[0:1], pos[1:2], pos[2:3]
    fx, fy, fz = pred[0:1], pred[1:2], pred[2:3]
    cx, cy, cz = _cross_rows(px, py, pz, fx, fy, fz)
    rsq = px * px + py * py + pz * pz
    feats = jnp.concatenate(
        [fx, fy, fz, px, py, pz, cx, cy, cz, rsq,
         px * px, py * py, pz * pz, px * py, px * pz, py * pz], axis=0)

    ids = jax.lax.broadcasted_iota(jnp.int32, (1, N_TILE), 1) + t * N_TILE
    sw = sw_ref[0]
    ew = ew_ref[0]
    onehot = jnp.where((ids >= sw) & (ids < ew), 1.0, 0.0)
    part = jax.lax.dot_general(feats, onehot, (((1,), (1,)), ((), ())),
                               preferred_element_type=jnp.float32)

    bj = bj_ref[0]
    giota = jax.lax.broadcasted_iota(jnp.int32, (WIN, mom_ref.shape[1]), 1)
    eqw = jnp.where(giota == bj, 1.0, 0.0)
    contrib = jnp.dot(part, eqw, preferred_element_type=jnp.float32)

    @pl.when(t == 0)
    def _():
        mom_ref[...] = contrib

    @pl.when(t > 0)
    def _():
        mom_ref[...] += contrib

    @pl.when(t == nt - 1)
    def _():
        table_ref[...] = _solve_from_moments(mom_ref[...], nn_ref[...],
                                             cell_ref[...])


SC_NC = 2
SC_NS = 16
SC_L = 16


def _apply_sc_body(pred_hbm, pos_hbm, tab_hbm, seg_hbm, out_hbm,
                   fxv, fyv, fzv, pxv, pyv, pzv, segv, tabv, outv):
    npw = segv.shape[0]
    n = pred_hbm.shape[0] // 3
    wid = lax.axis_index("s") * SC_NC + lax.axis_index("c")
    base = wid * npw
    pltpu.sync_copy(pred_hbm.at[pl.ds(base, npw)], fxv)
    pltpu.sync_copy(pred_hbm.at[pl.ds(n + base, npw)], fyv)
    pltpu.sync_copy(pred_hbm.at[pl.ds(2 * n + base, npw)], fzv)
    pltpu.sync_copy(pos_hbm.at[pl.ds(base, npw)], pxv)
    pltpu.sync_copy(pos_hbm.at[pl.ds(n + base, npw)], pyv)
    pltpu.sync_copy(pos_hbm.at[pl.ds(2 * n + base, npw)], pzv)
    pltpu.sync_copy(seg_hbm.at[pl.ds(base, npw)], segv)
    pltpu.sync_copy(tab_hbm, tabv)

    lane = lax.iota(jnp.int32, SC_L)

    nb = tab_hbm.shape[0] // 16

    def chunk(k, _):
        off = k * SC_L
        idx = segv[pl.ds(off, SC_L)]
        def gat(comp):
            return plsc.load_gather(tabv, [idx + comp * nb])
        mx, my, mz = gat(0), gat(1), gat(2)
        ox_, oy_, oz_ = gat(3), gat(4), gat(5)
        ux, uy, uz = gat(6), gat(7), gat(8)
        rx = pxv[pl.ds(off, SC_L)] - ox_
        ry = pyv[pl.ds(off, SC_L)] - oy_
        rz = pzv[pl.ds(off, SC_L)] - oz_
        vx = fxv[pl.ds(off, SC_L)] - mx + (ry * uz - rz * uy)
        vy = fyv[pl.ds(off, SC_L)] - my + (rz * ux - rx * uz)
        vz = fzv[pl.ds(off, SC_L)] - mz + (rx * uy - ry * ux)
        rows = (lane + off) * 3
        plsc.store_scatter(outv, [rows], vx)
        plsc.store_scatter(outv, [rows + 1], vy)
        plsc.store_scatter(outv, [rows + 2], vz)
        return _

    lax.fori_loop(0, npw // SC_L, chunk, None)
    pltpu.sync_copy(outv, out_hbm.at[pl.ds(3 * base, 3 * npw)])


def _apply_sc(pred_t, pos_t, table, seg):
    n = pred_t.shape[1]
    npw = n // (SC_NC * SC_NS)
    mesh = plsc.VectorSubcoreMesh(core_axis_name="c", subcore_axis_name="s")
    run = pl.kernel(
        _apply_sc_body,
        mesh=mesh,
        compiler_params=pltpu.CompilerParams(
            needs_layout_passes=False, use_tc_tiling_on_sc=False,
            skip_device_barrier=True),
        out_type=jax.ShapeDtypeStruct((3 * n,), jnp.float32),
        scratch_types=[
            pltpu.VMEM((npw,), jnp.float32),
            pltpu.VMEM((npw,), jnp.float32),
            pltpu.VMEM((npw,), jnp.float32),
            pltpu.VMEM((npw,), jnp.float32),
            pltpu.VMEM((npw,), jnp.float32),
            pltpu.VMEM((npw,), jnp.float32),
            pltpu.VMEM((npw,), jnp.int32),
            pltpu.VMEM((16 * 512,), jnp.float32),
            pltpu.VMEM((3 * npw,), jnp.float32),
        ],
    )
    return run(pred_t.reshape(3 * n), pos_t.reshape(3 * n),
               table.reshape(-1), seg).reshape(n, 3)


def _apply_body(pred_ref, pos_ref, sw_ref, ew_ref, bj_ref, table_ref, out_ref):
    t = pl.program_id(0)
    bj = bj_ref[0]
    giota = jax.lax.broadcasted_iota(jnp.int32, (512, 1), 0)
    eq = jnp.where(giota == bj, 1.0, 0.0)
    twin = jnp.dot(table_ref[...], eq, preferred_element_type=jnp.float32)

    ids = jax.lax.broadcasted_iota(jnp.int32, (1, N_TILE), 1) + t * N_TILE
    sw = sw_ref[0]
    ew = ew_ref[0]
    onehot = jnp.where((ids >= sw) & (ids < ew), 1.0, 0.0)
    vals = jnp.dot(twin, onehot, preferred_element_type=jnp.float32)

    pred = pred_ref[...]
    pos = pos_ref[...]
    rx = pos[0:1] - vals[3:4]
    ry = pos[1:2] - vals[4:5]
    rz = pos[2:3] - vals[5:6]
    dx, dy, dz = _cross_rows(rx, ry, rz, vals[6:7], vals[7:8], vals[8:9])
    ox = pred[0:1] - vals[0:1] + dx
    oy = pred[1:2] - vals[1:2] + dy
    oz = pred[2:3] - vals[2:3] + dz
    out_ref[...] = jnp.concatenate([ox, oy, oz], axis=0)


def kernel(x, positions, cell, n_node, W1, b1, W2, b2):
    N = x.shape[0]
    B = n_node.shape[0]
    T = N // N_TILE

    nn = n_node.astype(jnp.int32)
    ends = jnp.cumsum(nn)
    starts = ends - nn
    tile_starts = jnp.arange(T, dtype=jnp.int32) * N_TILE
    base = jnp.searchsorted(ends, tile_starts, side='right').astype(jnp.int32)
    win = base[:, None] + jnp.arange(WIN, dtype=jnp.int32)[None, :]
    valid = win < B
    winc = jnp.clip(win, 0, B - 1)
    s_w = jnp.where(valid, starts[winc], N).astype(jnp.int32)
    e_w = jnp.where(valid, ends[winc], N).astype(jnp.int32)
    bj = jnp.where(valid, win, -1).astype(jnp.int32)
    sw3 = s_w.reshape(T, WIN, 1)
    ew3 = e_w.reshape(T, WIN, 1)
    bjc = bj.reshape(T, WIN, 1)
    bjr = bj.reshape(T, 1, WIN)
    nnf = n_node.astype(jnp.float32).reshape(1, B)
    cell_t = cell.reshape(B, 9).T
    pos_t = positions.T

    pred_t, table = pl.pallas_call(
        _mlp_moments_body,
        grid=(T,),
        in_specs=[
            pl.BlockSpec((N_TILE, 128), lambda t: (t, 0)),
            pl.BlockSpec((128, 128), lambda t: (0, 0)),
            pl.BlockSpec((1, 128), lambda t: (0, 0)),
            pl.BlockSpec((128, 3), lambda t: (0, 0)),
            pl.BlockSpec((3, 1), lambda t: (0, 0)),
            pl.BlockSpec((3, N_TILE), lambda t: (0, t)),
            pl.BlockSpec((1, WIN, 1), lambda t: (t, 0, 0)),
            pl.BlockSpec((1, WIN, 1), lambda t: (t, 0, 0)),
            pl.BlockSpec((1, WIN, 1), lambda t: (t, 0, 0)),
            pl.BlockSpec((1, B), lambda t: (0, 0)),
            pl.BlockSpec((9, B), lambda t: (0, 0)),
        ],
        out_specs=[
            pl.BlockSpec((3, N_TILE), lambda t: (0, t)),
            pl.BlockSpec((16, B), lambda t: (0, 0)),
        ],
        out_shape=[
            jax.ShapeDtypeStruct((3, N), jnp.float32),
            jax.ShapeDtypeStruct((16, B), jnp.float32),
        ],
        scratch_shapes=[pltpu.VMEM((16, B), jnp.float32)],
        compiler_params=pltpu.CompilerParams(
            dimension_semantics=("arbitrary",)),
    )(x, W1, b1.reshape(1, 128), W2, b2.reshape(3, 1), pos_t, sw3, ew3, bjc,
      nnf, cell_t)

    seg = jnp.searchsorted(ends, jnp.arange(N, dtype=jnp.int32),
                           side='right').astype(jnp.int32)
    return _apply_sc(pred_t, pos_t, table, seg)


# speedup vs baseline: 42.5112x; 42.5112x over previous
import jax
import jax.numpy as jnp
from jax import lax
from jax.experimental import pallas as pl
from jax.experimental.pallas import tpu as pltpu
from jax.experimental.pallas import tpu_sc as plsc

N_TILE = 2048
WIN = 32


def _cross_rows(ax, ay, az, bx, by, bz):
    return (ay * bz - az * by, az * bx - ax * bz, ax * by - ay * bx)


def _solve_from_moments(mom, nn, cell):
    ninv = 1.0 / nn
    ax_, ay_, az_ = mom[0:1], mom[1:2], mom[2:3]
    px_, py_, pz_ = mom[3:4], mom[4:5], mom[5:6]
    cx_, cy_, cz_ = mom[6:7], mom[7:8], mom[8:9]
    q = mom[9:10]
    oxx, oyy, ozz = mom[10:11], mom[11:12], mom[12:13]
    oxy, oxz, oyz = mom[13:14], mom[14:15], mom[15:16]

    mean_x, mean_y, mean_z = ax_ * ninv, ay_ * ninv, az_ * ninv
    com_x, com_y, com_z = px_ * ninv, py_ * ninv, pz_ * ninv
    kx, ky, kz = _cross_rows(px_, py_, pz_, ax_, ay_, az_)
    tx = cx_ - kx * ninv
    ty = cy_ - ky * ninv
    tz = cz_ - kz * ninv
    s = q - (px_ * px_ + py_ * py_ + pz_ * pz_) * ninv
    a = oxx - px_ * px_ * ninv - s
    d = oyy - py_ * py_ * ninv - s
    f = ozz - pz_ * pz_ * ninv - s
    b = oxy - px_ * py_ * ninv
    c = oxz - px_ * pz_ * ninv
    e = oyz - py_ * pz_ * ninv

    det = a * (d * f - e * e) - b * (b * f - e * c) + c * (b * e - d * c)
    dinv = 1.0 / det
    i00 = d * f - e * e
    i01 = c * e - b * f
    i02 = b * e - c * d
    i11 = a * f - c * c
    i12 = b * c - a * e
    i22 = a * d - b * b
    mux = -(i00 * tx + i01 * ty + i02 * tz) * dinv
    muy = -(i01 * tx + i11 * ty + i12 * tz) * dinv
    muz = -(i02 * tx + i12 * ty + i22 * tz) * dinv
    rx = tx + a * mux + b * muy + c * muz
    ry = ty + b * mux + d * muy + e * muz
    rz = tz + c * mux + e * muy + f * muz
    mux = mux - (i00 * rx + i01 * ry + i02 * rz) * dinv
    muy = muy - (i01 * rx + i11 * ry + i12 * rz) * dinv
    muz = muz - (i02 * rx + i12 * ry + i22 * rz) * dinv

    nopbc = jnp.all(cell == 0.0, axis=0, keepdims=True)
    zero = jnp.zeros_like(mux)
    mux = jnp.where(nopbc, mux, zero)
    muy = jnp.where(nopbc, muy, zero)
    muz = jnp.where(nopbc, muz, zero)

    return jnp.concatenate(
        [mean_x, mean_y, mean_z, com_x, com_y, com_z, mux, muy, muz,
         zero, zero, zero, zero, zero, zero, zero], axis=0)


def _mlp_moments_body(x_ref, w1_ref, b1_ref, w2_ref, b2_ref, pos_ref,
                      sw_ref, ew_ref, bj_ref, nn_ref, cell_ref,
                      pred_ref, table_ref, seg_ref, mom_ref):
    t = pl.program_id(0)
    nt = pl.num_programs(0)
    h = jax.nn.gelu(jnp.dot(x_ref[...], w1_ref[...],
                            preferred_element_type=jnp.float32) + b1_ref[...])
    pred = jax.lax.dot_general(w2_ref[...], h, (((0,), (1,)), ((), ())),
                               preferred_element_type=jnp.float32) + b2_ref[...]
    pred_ref[...] = pred

    pos = pos_ref[...]
    px, py, pz = pos[0:1], pos[1:2], pos[2:3]
    fx, fy, fz = pred[0:1], pred[1:2], pred[2:3]
    cx, cy, cz = _cross_rows(px, py, pz, fx, fy, fz)
    rsq = px * px + py * py + pz * pz
    feats = jnp.concatenate(
        [fx, fy, fz, px, py, pz, cx, cy, cz, rsq,
         px * px, py * py, pz * pz, px * py, px * pz, py * pz], axis=0)

    ids = jax.lax.broadcasted_iota(jnp.int32, (1, N_TILE), 1) + t * N_TILE
    sw = sw_ref[0]
    ew = ew_ref[0]
    onehot = jnp.where((ids >= sw) & (ids < ew), 1.0, 0.0)
    part = jax.lax.dot_general(feats, onehot, (((1,), (1,)), ((), ())),
                               preferred_element_type=jnp.float32)

    bj = bj_ref[0]
    giota = jax.lax.broadcasted_iota(jnp.int32, (WIN, mom_ref.shape[1]), 1)
    eqw = jnp.where(giota == bj, 1.0, 0.0)
    contrib = jnp.dot(part, eqw, preferred_element_type=jnp.float32)

    segf = jax.lax.dot_general(bj.astype(jnp.float32), onehot,
                               (((0,), (0,)), ((), ())),
                               preferred_element_type=jnp.float32)
    seg_ref[...] = segf.astype(jnp.int32)

    @pl.when(t == 0)
    def _():
        mom_ref[...] = contrib

    @pl.when(t > 0)
    def _():
        mom_ref[...] += contrib

    @pl.when(t == nt - 1)
    def _():
        table_ref[...] = _solve_from_moments(mom_ref[...], nn_ref[...],
                                             cell_ref[...])


SC_NC = 2
SC_NS = 16
SC_L = 16


def _apply_sc_body(pred_hbm, pos_hbm, tab_hbm, seg_hbm, out_hbm,
                   fxv, fyv, fzv, pxv, pyv, pzv, segv, tabv, outv):
    npw = segv.shape[0]
    n = pred_hbm.shape[0] // 3
    wid = lax.axis_index("s") * SC_NC + lax.axis_index("c")
    base = wid * npw
    pltpu.sync_copy(pred_hbm.at[pl.ds(base, npw)], fxv)
    pltpu.sync_copy(pred_hbm.at[pl.ds(n + base, npw)], fyv)
    pltpu.sync_copy(pred_hbm.at[pl.ds(2 * n + base, npw)], fzv)
    pltpu.sync_copy(pos_hbm.at[pl.ds(base, npw)], pxv)
    pltpu.sync_copy(pos_hbm.at[pl.ds(n + base, npw)], pyv)
    pltpu.sync_copy(pos_hbm.at[pl.ds(2 * n + base, npw)], pzv)
    pltpu.sync_copy(seg_hbm.at[pl.ds(base, npw)], segv)
    pltpu.sync_copy(tab_hbm, tabv)

    lane = lax.iota(jnp.int32, SC_L)

    nb = tab_hbm.shape[0] // 16

    @plsc.parallel_loop(0, npw // SC_L, unroll=8)
    def chunk(k):
        off = k * SC_L
        idx = segv[pl.ds(off, SC_L)]
        def gat(comp):
            return plsc.load_gather(tabv, [idx + comp * nb])
        mx, my, mz = gat(0), gat(1), gat(2)
        ox_, oy_, oz_ = gat(3), gat(4), gat(5)
        ux, uy, uz = gat(6), gat(7), gat(8)
        rx = pxv[pl.ds(off, SC_L)] - ox_
        ry = pyv[pl.ds(off, SC_L)] - oy_
        rz = pzv[pl.ds(off, SC_L)] - oz_
        vx = fxv[pl.ds(off, SC_L)] - mx + (ry * uz - rz * uy)
        vy = fyv[pl.ds(off, SC_L)] - my + (rz * ux - rx * uz)
        vz = fzv[pl.ds(off, SC_L)] - mz + (rx * uy - ry * ux)
        rows = (lane + off) * 3
        plsc.store_scatter(outv, [rows], vx)
        plsc.store_scatter(outv, [rows + 1], vy)
        plsc.store_scatter(outv, [rows + 2], vz)

    pltpu.sync_copy(outv, out_hbm.at[pl.ds(3 * base, 3 * npw)])


def _apply_sc(pred_t, pos_t, table, seg):
    n = pred_t.shape[1]
    npw = n // (SC_NC * SC_NS)
    mesh = plsc.VectorSubcoreMesh(core_axis_name="c", subcore_axis_name="s")
    run = pl.kernel(
        _apply_sc_body,
        mesh=mesh,
        compiler_params=pltpu.CompilerParams(
            needs_layout_passes=False, use_tc_tiling_on_sc=False,
            skip_device_barrier=True),
        out_type=jax.ShapeDtypeStruct((3 * n,), jnp.float32),
        scratch_types=[
            pltpu.VMEM((npw,), jnp.float32),
            pltpu.VMEM((npw,), jnp.float32),
            pltpu.VMEM((npw,), jnp.float32),
            pltpu.VMEM((npw,), jnp.float32),
            pltpu.VMEM((npw,), jnp.float32),
            pltpu.VMEM((npw,), jnp.float32),
            pltpu.VMEM((npw,), jnp.int32),
            pltpu.VMEM((16 * 512,), jnp.float32),
            pltpu.VMEM((3 * npw,), jnp.float32),
        ],
    )
    return run(pred_t.reshape(3 * n), pos_t.reshape(3 * n),
               table.reshape(-1), seg).reshape(n, 3)


def _apply_body(pred_ref, pos_ref, sw_ref, ew_ref, bj_ref, table_ref, out_ref):
    t = pl.program_id(0)
    bj = bj_ref[0]
    giota = jax.lax.broadcasted_iota(jnp.int32, (512, 1), 0)
    eq = jnp.where(giota == bj, 1.0, 0.0)
    twin = jnp.dot(table_ref[...], eq, preferred_element_type=jnp.float32)

    ids = jax.lax.broadcasted_iota(jnp.int32, (1, N_TILE), 1) + t * N_TILE
    sw = sw_ref[0]
    ew = ew_ref[0]
    onehot = jnp.where((ids >= sw) & (ids < ew), 1.0, 0.0)
    vals = jnp.dot(twin, onehot, preferred_element_type=jnp.float32)

    pred = pred_ref[...]
    pos = pos_ref[...]
    rx = pos[0:1] - vals[3:4]
    ry = pos[1:2] - vals[4:5]
    rz = pos[2:3] - vals[5:6]
    dx, dy, dz = _cross_rows(rx, ry, rz, vals[6:7], vals[7:8], vals[8:9])
    ox = pred[0:1] - vals[0:1] + dx
    oy = pred[1:2] - vals[1:2] + dy
    oz = pred[2:3] - vals[2:3] + dz
    out_ref[...] = jnp.concatenate([ox, oy, oz], axis=0)


def kernel(x, positions, cell, n_node, W1, b1, W2, b2):
    N = x.shape[0]
    B = n_node.shape[0]
    T = N // N_TILE

    nn = n_node.astype(jnp.int32)
    ends = jnp.cumsum(nn)
    starts = ends - nn
    tile_starts = jnp.arange(T, dtype=jnp.int32) * N_TILE
    base = jnp.searchsorted(ends, tile_starts, side='right').astype(jnp.int32)
    win = base[:, None] + jnp.arange(WIN, dtype=jnp.int32)[None, :]
    valid = win < B
    winc = jnp.clip(win, 0, B - 1)
    s_w = jnp.where(valid, starts[winc], N).astype(jnp.int32)
    e_w = jnp.where(valid, ends[winc], N).astype(jnp.int32)
    bj = jnp.where(valid, win, -1).astype(jnp.int32)
    sw3 = s_w.reshape(T, WIN, 1)
    ew3 = e_w.reshape(T, WIN, 1)
    bjc = bj.reshape(T, WIN, 1)
    bjr = bj.reshape(T, 1, WIN)
    nnf = n_node.astype(jnp.float32).reshape(1, B)
    cell_t = cell.reshape(B, 9).T
    pos_t = positions.T

    pred_t, table, seg = pl.pallas_call(
        _mlp_moments_body,
        grid=(T,),
        in_specs=[
            pl.BlockSpec((N_TILE, 128), lambda t: (t, 0)),
            pl.BlockSpec((128, 128), lambda t: (0, 0)),
            pl.BlockSpec((1, 128), lambda t: (0, 0)),
            pl.BlockSpec((128, 3), lambda t: (0, 0)),
            pl.BlockSpec((3, 1), lambda t: (0, 0)),
            pl.BlockSpec((3, N_TILE), lambda t: (0, t)),
            pl.BlockSpec((1, WIN, 1), lambda t: (t, 0, 0)),
            pl.BlockSpec((1, WIN, 1), lambda t: (t, 0, 0)),
            pl.BlockSpec((1, WIN, 1), lambda t: (t, 0, 0)),
            pl.BlockSpec((1, B), lambda t: (0, 0)),
            pl.BlockSpec((9, B), lambda t: (0, 0)),
        ],
        out_specs=[
            pl.BlockSpec((3, N_TILE), lambda t: (0, t)),
            pl.BlockSpec((16, B), lambda t: (0, 0)),
            pl.BlockSpec((1, N_TILE), lambda t: (0, t)),
        ],
        out_shape=[
            jax.ShapeDtypeStruct((3, N), jnp.float32),
            jax.ShapeDtypeStruct((16, B), jnp.float32),
            jax.ShapeDtypeStruct((1, N), jnp.int32),
        ],
        scratch_shapes=[pltpu.VMEM((16, B), jnp.float32)],
        compiler_params=pltpu.CompilerParams(
            dimension_semantics=("arbitrary",)),
    )(x, W1, b1.reshape(1, 128), W2, b2.reshape(3, 1), pos_t, sw3, ew3, bjc,
      nnf, cell_t)

    return _apply_sc(pred_t, pos_t, table, seg.reshape(N))


# speedup vs baseline: 42.6243x; 1.0027x over previous
import jax
import jax.numpy as jnp
from jax import lax
from jax.experimental import pallas as pl
from jax.experimental.pallas import tpu as pltpu
from jax.experimental.pallas import tpu_sc as plsc

N_TILE = 2048
WIN = 32


def _cross_rows(ax, ay, az, bx, by, bz):
    return (ay * bz - az * by, az * bx - ax * bz, ax * by - ay * bx)


def _solve_from_moments(mom, nn, cell):
    ninv = 1.0 / nn
    ax_, ay_, az_ = mom[0:1], mom[1:2], mom[2:3]
    px_, py_, pz_ = mom[3:4], mom[4:5], mom[5:6]
    cx_, cy_, cz_ = mom[6:7], mom[7:8], mom[8:9]
    q = mom[9:10]
    oxx, oyy, ozz = mom[10:11], mom[11:12], mom[12:13]
    oxy, oxz, oyz = mom[13:14], mom[14:15], mom[15:16]

    mean_x, mean_y, mean_z = ax_ * ninv, ay_ * ninv, az_ * ninv
    com_x, com_y, com_z = px_ * ninv, py_ * ninv, pz_ * ninv
    kx, ky, kz = _cross_rows(px_, py_, pz_, ax_, ay_, az_)
    tx = cx_ - kx * ninv
    ty = cy_ - ky * ninv
    tz = cz_ - kz * ninv
    s = q - (px_ * px_ + py_ * py_ + pz_ * pz_) * ninv
    a = oxx - px_ * px_ * ninv - s
    d = oyy - py_ * py_ * ninv - s
    f = ozz - pz_ * pz_ * ninv - s
    b = oxy - px_ * py_ * ninv
    c = oxz - px_ * pz_ * ninv
    e = oyz - py_ * pz_ * ninv

    det = a * (d * f - e * e) - b * (b * f - e * c) + c * (b * e - d * c)
    dinv = 1.0 / det
    i00 = d * f - e * e
    i01 = c * e - b * f
    i02 = b * e - c * d
    i11 = a * f - c * c
    i12 = b * c - a * e
    i22 = a * d - b * b
    mux = -(i00 * tx + i01 * ty + i02 * tz) * dinv
    muy = -(i01 * tx + i11 * ty + i12 * tz) * dinv
    muz = -(i02 * tx + i12 * ty + i22 * tz) * dinv
    rx = tx + a * mux + b * muy + c * muz
    ry = ty + b * mux + d * muy + e * muz
    rz = tz + c * mux + e * muy + f * muz
    mux = mux - (i00 * rx + i01 * ry + i02 * rz) * dinv
    muy = muy - (i01 * rx + i11 * ry + i12 * rz) * dinv
    muz = muz - (i02 * rx + i12 * ry + i22 * rz) * dinv

    nopbc = jnp.all(cell == 0.0, axis=0, keepdims=True)
    zero = jnp.zeros_like(mux)
    mux = jnp.where(nopbc, mux, zero)
    muy = jnp.where(nopbc, muy, zero)
    muz = jnp.where(nopbc, muz, zero)

    return jnp.concatenate(
        [mean_x, mean_y, mean_z, com_x, com_y, com_z, mux, muy, muz,
         zero, zero, zero, zero, zero, zero, zero], axis=0)


def _mlp_moments_body(x_ref, w1_ref, b1_ref, w2_ref, b2_ref, pos_ref,
                      sw_ref, ew_ref, bj_ref, nn_ref, cell_ref,
                      pred_ref, table_ref, seg_ref, mom_ref):
    t = pl.program_id(0)
    nt = pl.num_programs(0)
    h = jax.nn.gelu(jnp.dot(x_ref[...], w1_ref[...],
                            preferred_element_type=jnp.float32) + b1_ref[...])
    pred = jax.lax.dot_general(w2_ref[...], h, (((0,), (1,)), ((), ())),
                               preferred_element_type=jnp.float32) + b2_ref[...]
    pred_ref[...] = pred

    pos = pos_ref[...]
    px, py, pz = pos[0:1], pos[1:2], pos[2:3]
    fx, fy, fz = pred[0:1], pred[1:2], pred[2:3]
    cx, cy, cz = _cross_rows(px, py, pz, fx, fy, fz)
    rsq = px * px + py * py + pz * pz
    feats = jnp.concatenate(
        [fx, fy, fz, px, py, pz, cx, cy, cz, rsq,
         px * px, py * py, pz * pz, px * py, px * pz, py * pz], axis=0)

    ids = jax.lax.broadcasted_iota(jnp.int32, (1, N_TILE), 1) + t * N_TILE
    sw = sw_ref[0]
    ew = ew_ref[0]
    onehot = jnp.where((ids >= sw) & (ids < ew), 1.0, 0.0)
    part = jax.lax.dot_general(feats, onehot, (((1,), (1,)), ((), ())),
                               preferred_element_type=jnp.float32)

    bj = bj_ref[0]
    giota = jax.lax.broadcasted_iota(jnp.int32, (WIN, mom_ref.shape[1]), 1)
    eqw = jnp.where(giota == bj, 1.0, 0.0)
    contrib = jnp.dot(part, eqw, preferred_element_type=jnp.float32)

    inwin = (ids >= sw) & (ids < ew)
    seg_ref[...] = jnp.sum(jnp.where(inwin, bj, 0), axis=0, keepdims=True)

    @pl.when(t == 0)
    def _():
        mom_ref[...] = contrib

    @pl.when(t > 0)
    def _():
        mom_ref[...] += contrib

    @pl.when(t == nt - 1)
    def _():
        table_ref[...] = _solve_from_moments(mom_ref[...], nn_ref[...],
                                             cell_ref[...])


SC_NC = 2
SC_NS = 16
SC_L = 16


def _apply_sc_body(pred_hbm, pos_hbm, tab_hbm, seg_hbm, out_hbm,
                   fxv, fyv, fzv, pxv, pyv, pzv, segv, tabv, outv):
    npw = segv.shape[0]
    n = pred_hbm.shape[0] // 3
    wid = lax.axis_index("s") * SC_NC + lax.axis_index("c")
    base = wid * npw
    pltpu.sync_copy(pred_hbm.at[pl.ds(base, npw)], fxv)
    pltpu.sync_copy(pred_hbm.at[pl.ds(n + base, npw)], fyv)
    pltpu.sync_copy(pred_hbm.at[pl.ds(2 * n + base, npw)], fzv)
    pltpu.sync_copy(pos_hbm.at[pl.ds(base, npw)], pxv)
    pltpu.sync_copy(pos_hbm.at[pl.ds(n + base, npw)], pyv)
    pltpu.sync_copy(pos_hbm.at[pl.ds(2 * n + base, npw)], pzv)
    pltpu.sync_copy(seg_hbm.at[pl.ds(base, npw)], segv)
    pltpu.sync_copy(tab_hbm, tabv)

    lane = lax.iota(jnp.int32, SC_L)

    nb = tab_hbm.shape[0] // 16

    @plsc.parallel_loop(0, npw // SC_L, unroll=8)
    def chunk(k):
        off = k * SC_L
        idx = segv[pl.ds(off, SC_L)]
        def gat(comp):
            return plsc.load_gather(tabv, [idx + comp * nb])
        mx, my, mz = gat(0), gat(1), gat(2)
        ox_, oy_, oz_ = gat(3), gat(4), gat(5)
        ux, uy, uz = gat(6), gat(7), gat(8)
        rx = pxv[pl.ds(off, SC_L)] - ox_
        ry = pyv[pl.ds(off, SC_L)] - oy_
        rz = pzv[pl.ds(off, SC_L)] - oz_
        vx = fxv[pl.ds(off, SC_L)] - mx + (ry * uz - rz * uy)
        vy = fyv[pl.ds(off, SC_L)] - my + (rz * ux - rx * uz)
        vz = fzv[pl.ds(off, SC_L)] - mz + (rx * uy - ry * ux)
        rows = (lane + off) * 3
        plsc.store_scatter(outv, [rows], vx)
        plsc.store_scatter(outv, [rows + 1], vy)
        plsc.store_scatter(outv, [rows + 2], vz)

    pltpu.sync_copy(outv, out_hbm.at[pl.ds(3 * base, 3 * npw)])


def _apply_sc(pred_t, pos_t, table, seg):
    n = pred_t.shape[1]
    npw = n // (SC_NC * SC_NS)
    mesh = plsc.VectorSubcoreMesh(core_axis_name="c", subcore_axis_name="s")
    run = pl.kernel(
        _apply_sc_body,
        mesh=mesh,
        compiler_params=pltpu.CompilerParams(
            needs_layout_passes=False, use_tc_tiling_on_sc=False,
            skip_device_barrier=True),
        out_type=jax.ShapeDtypeStruct((3 * n,), jnp.float32),
        scratch_types=[
            pltpu.VMEM((npw,), jnp.float32),
            pltpu.VMEM((npw,), jnp.float32),
            pltpu.VMEM((npw,), jnp.float32),
            pltpu.VMEM((npw,), jnp.float32),
            pltpu.VMEM((npw,), jnp.float32),
            pltpu.VMEM((npw,), jnp.float32),
            pltpu.VMEM((npw,), jnp.int32),
            pltpu.VMEM((16 * 512,), jnp.float32),
            pltpu.VMEM((3 * npw,), jnp.float32),
        ],
    )
    return run(pred_t.reshape(3 * n), pos_t.reshape(3 * n),
               table.reshape(-1), seg).reshape(n, 3)


def _apply_body(pred_ref, pos_ref, sw_ref, ew_ref, bj_ref, table_ref, out_ref):
    t = pl.program_id(0)
    bj = bj_ref[0]
    giota = jax.lax.broadcasted_iota(jnp.int32, (512, 1), 0)
    eq = jnp.where(giota == bj, 1.0, 0.0)
    twin = jnp.dot(table_ref[...], eq, preferred_element_type=jnp.float32)

    ids = jax.lax.broadcasted_iota(jnp.int32, (1, N_TILE), 1) + t * N_TILE
    sw = sw_ref[0]
    ew = ew_ref[0]
    onehot = jnp.where((ids >= sw) & (ids < ew), 1.0, 0.0)
    vals = jnp.dot(twin, onehot, preferred_element_type=jnp.float32)

    pred = pred_ref[...]
    pos = pos_ref[...]
    rx = pos[0:1] - vals[3:4]
    ry = pos[1:2] - vals[4:5]
    rz = pos[2:3] - vals[5:6]
    dx, dy, dz = _cross_rows(rx, ry, rz, vals[6:7], vals[7:8], vals[8:9])
    ox = pred[0:1] - vals[0:1] + dx
    oy = pred[1:2] - vals[1:2] + dy
    oz = pred[2:3] - vals[2:3] + dz
    out_ref[...] = jnp.concatenate([ox, oy, oz], axis=0)


def kernel(x, positions, cell, n_node, W1, b1, W2, b2):
    N = x.shape[0]
    B = n_node.shape[0]
    T = N // N_TILE

    nn = n_node.astype(jnp.int32)
    ends = jnp.cumsum(nn)
    starts = ends - nn
    tile_starts = jnp.arange(T, dtype=jnp.int32) * N_TILE
    base = jnp.searchsorted(ends, tile_starts, side='right').astype(jnp.int32)
    win = base[:, None] + jnp.arange(WIN, dtype=jnp.int32)[None, :]
    valid = win < B
    winc = jnp.clip(win, 0, B - 1)
    s_w = jnp.where(valid, starts[winc], N).astype(jnp.int32)
    e_w = jnp.where(valid, ends[winc], N).astype(jnp.int32)
    bj = jnp.where(valid, win, -1).astype(jnp.int32)
    sw3 = s_w.reshape(T, WIN, 1)
    ew3 = e_w.reshape(T, WIN, 1)
    bjc = bj.reshape(T, WIN, 1)
    bjr = bj.reshape(T, 1, WIN)
    nnf = n_node.astype(jnp.float32).reshape(1, B)
    cell_t = cell.reshape(B, 9).T
    pos_t = positions.T

    pred_t, table, seg = pl.pallas_call(
        _mlp_moments_body,
        grid=(T,),
        in_specs=[
            pl.BlockSpec((N_TILE, 128), lambda t: (t, 0)),
            pl.BlockSpec((128, 128), lambda t: (0, 0)),
            pl.BlockSpec((1, 128), lambda t: (0, 0)),
            pl.BlockSpec((128, 3), lambda t: (0, 0)),
            pl.BlockSpec((3, 1), lambda t: (0, 0)),
            pl.BlockSpec((3, N_TILE), lambda t: (0, t)),
            pl.BlockSpec((1, WIN, 1), lambda t: (t, 0, 0)),
            pl.BlockSpec((1, WIN, 1), lambda t: (t, 0, 0)),
            pl.BlockSpec((1, WIN, 1), lambda t: (t, 0, 0)),
            pl.BlockSpec((1, B), lambda t: (0, 0)),
            pl.BlockSpec((9, B), lambda t: (0, 0)),
        ],
        out_specs=[
            pl.BlockSpec((3, N_TILE), lambda t: (0, t)),
            pl.BlockSpec((16, B), lambda t: (0, 0)),
            pl.BlockSpec((1, N_TILE), lambda t: (0, t)),
        ],
        out_shape=[
            jax.ShapeDtypeStruct((3, N), jnp.float32),
            jax.ShapeDtypeStruct((16, B), jnp.float32),
            jax.ShapeDtypeStruct((1, N), jnp.int32),
        ],
        scratch_shapes=[pltpu.VMEM((16, B), jnp.float32)],
        compiler_params=pltpu.CompilerParams(
            dimension_semantics=("arbitrary",)),
    )(x, W1, b1.reshape(1, 128), W2, b2.reshape(3, 1), pos_t, sw3, ew3, bjc,
      nnf, cell_t)

    return _apply_sc(pred_t, pos_t, table, seg.reshape(N))


# speedup vs baseline: 68.2963x; 1.6023x over previous
import jax
import jax.numpy as jnp
from jax import lax
from jax.experimental import pallas as pl
from jax.experimental.pallas import tpu as pltpu
from jax.experimental.pallas import tpu_sc as plsc

N_TILE = 2048
WIN = 32


def _cross_rows(ax, ay, az, bx, by, bz):
    return (ay * bz - az * by, az * bx - ax * bz, ax * by - ay * bx)


def _solve_from_moments(mom, nn, cell):
    ninv = 1.0 / nn
    ax_, ay_, az_ = mom[0:1], mom[1:2], mom[2:3]
    px_, py_, pz_ = mom[3:4], mom[4:5], mom[5:6]
    cx_, cy_, cz_ = mom[6:7], mom[7:8], mom[8:9]
    q = mom[9:10]
    oxx, oyy, ozz = mom[10:11], mom[11:12], mom[12:13]
    oxy, oxz, oyz = mom[13:14], mom[14:15], mom[15:16]

    mean_x, mean_y, mean_z = ax_ * ninv, ay_ * ninv, az_ * ninv
    com_x, com_y, com_z = px_ * ninv, py_ * ninv, pz_ * ninv
    kx, ky, kz = _cross_rows(px_, py_, pz_, ax_, ay_, az_)
    tx = cx_ - kx * ninv
    ty = cy_ - ky * ninv
    tz = cz_ - kz * ninv
    s = q - (px_ * px_ + py_ * py_ + pz_ * pz_) * ninv
    a = oxx - px_ * px_ * ninv - s
    d = oyy - py_ * py_ * ninv - s
    f = ozz - pz_ * pz_ * ninv - s
    b = oxy - px_ * py_ * ninv
    c = oxz - px_ * pz_ * ninv
    e = oyz - py_ * pz_ * ninv

    det = a * (d * f - e * e) - b * (b * f - e * c) + c * (b * e - d * c)
    dinv = 1.0 / det
    i00 = d * f - e * e
    i01 = c * e - b * f
    i02 = b * e - c * d
    i11 = a * f - c * c
    i12 = b * c - a * e
    i22 = a * d - b * b
    mux = -(i00 * tx + i01 * ty + i02 * tz) * dinv
    muy = -(i01 * tx + i11 * ty + i12 * tz) * dinv
    muz = -(i02 * tx + i12 * ty + i22 * tz) * dinv
    rx = tx + a * mux + b * muy + c * muz
    ry = ty + b * mux + d * muy + e * muz
    rz = tz + c * mux + e * muy + f * muz
    mux = mux - (i00 * rx + i01 * ry + i02 * rz) * dinv
    muy = muy - (i01 * rx + i11 * ry + i12 * rz) * dinv
    muz = muz - (i02 * rx + i12 * ry + i22 * rz) * dinv

    nopbc = jnp.all(cell == 0.0, axis=0, keepdims=True)
    zero = jnp.zeros_like(mux)
    mux = jnp.where(nopbc, mux, zero)
    muy = jnp.where(nopbc, muy, zero)
    muz = jnp.where(nopbc, muz, zero)

    return jnp.concatenate(
        [mean_x, mean_y, mean_z, com_x, com_y, com_z, mux, muy, muz,
         zero, zero, zero, zero, zero, zero, zero], axis=0)


def _mlp_moments_body(x_ref, w1_ref, b1_ref, w2_ref, b2_ref, pos_ref,
                      sw_ref, ew_ref, bj_ref, nn_ref, cell_ref,
                      pred_ref, table_ref, seg_ref, mom_ref):
    t = pl.program_id(0)
    nt = pl.num_programs(0)
    h = jax.nn.gelu(jnp.dot(x_ref[...], w1_ref[...],
                            preferred_element_type=jnp.float32) + b1_ref[...])
    pred = jax.lax.dot_general(w2_ref[...], h, (((0,), (1,)), ((), ())),
                               preferred_element_type=jnp.float32) + b2_ref[...]
    pred_ref[...] = pred

    pos = pos_ref[...]
    px, py, pz = pos[0:1], pos[1:2], pos[2:3]
    fx, fy, fz = pred[0:1], pred[1:2], pred[2:3]
    cx, cy, cz = _cross_rows(px, py, pz, fx, fy, fz)
    rsq = px * px + py * py + pz * pz
    feats = jnp.concatenate(
        [fx, fy, fz, px, py, pz, cx, cy, cz, rsq,
         px * px, py * py, pz * pz, px * py, px * pz, py * pz], axis=0)

    ids = jax.lax.broadcasted_iota(jnp.int32, (1, N_TILE), 1) + t * N_TILE
    sw = sw_ref[0]
    ew = ew_ref[0]
    onehot = jnp.where((ids >= sw) & (ids < ew), 1.0, 0.0)
    part = jax.lax.dot_general(feats, onehot, (((1,), (1,)), ((), ())),
                               preferred_element_type=jnp.float32)

    bj = bj_ref[0]
    giota = jax.lax.broadcasted_iota(jnp.int32, (WIN, mom_ref.shape[1]), 1)
    eqw = jnp.where(giota == bj, 1.0, 0.0)
    contrib = jnp.dot(part, eqw, preferred_element_type=jnp.float32)

    inwin = (ids >= sw) & (ids < ew)
    seg_ref[...] = jnp.sum(jnp.where(inwin, bj, 0), axis=0, keepdims=True)

    @pl.when(t == 0)
    def _():
        mom_ref[...] = contrib

    @pl.when(t > 0)
    def _():
        mom_ref[...] += contrib

    @pl.when(t == nt - 1)
    def _():
        table_ref[...] = _solve_from_moments(mom_ref[...], nn_ref[...],
                                             cell_ref[...])


SC_NC = 2
SC_NS = 16
SC_L = 16


def _apply_sc_body(pred_hbm, pos_hbm, tab_hbm, seg_hbm, out_hbm,
                   fxv, fyv, fzv, pxv, pyv, pzv, segv, tabv, outv):
    npw = segv.shape[0]
    n = pred_hbm.shape[0] // 3
    wid = lax.axis_index("s") * SC_NC + lax.axis_index("c")
    base = wid * npw
    pltpu.sync_copy(pred_hbm.at[pl.ds(base, npw)], fxv)
    pltpu.sync_copy(pred_hbm.at[pl.ds(n + base, npw)], fyv)
    pltpu.sync_copy(pred_hbm.at[pl.ds(2 * n + base, npw)], fzv)
    pltpu.sync_copy(pos_hbm.at[pl.ds(base, npw)], pxv)
    pltpu.sync_copy(pos_hbm.at[pl.ds(n + base, npw)], pyv)
    pltpu.sync_copy(pos_hbm.at[pl.ds(2 * n + base, npw)], pzv)
    pltpu.sync_copy(seg_hbm.at[pl.ds(base, npw)], segv)
    pltpu.sync_copy(tab_hbm, tabv)

    nb = tab_hbm.shape[0] // 16

    @plsc.parallel_loop(0, npw // SC_L, unroll=8)
    def chunk(k):
        off = k * SC_L
        idx = segv[pl.ds(off, SC_L)]
        def gat(comp):
            return plsc.load_gather(tabv, [idx + comp * nb])
        mx, my, mz = gat(0), gat(1), gat(2)
        ox_, oy_, oz_ = gat(3), gat(4), gat(5)
        ux, uy, uz = gat(6), gat(7), gat(8)
        rx = pxv[pl.ds(off, SC_L)] - ox_
        ry = pyv[pl.ds(off, SC_L)] - oy_
        rz = pzv[pl.ds(off, SC_L)] - oz_
        outv[pl.ds(off, SC_L)] = fxv[pl.ds(off, SC_L)] - mx + (ry * uz - rz * uy)
        outv[pl.ds(npw + off, SC_L)] = fyv[pl.ds(off, SC_L)] - my + (rz * ux - rx * uz)
        outv[pl.ds(2 * npw + off, SC_L)] = fzv[pl.ds(off, SC_L)] - mz + (rx * uy - ry * ux)

    pltpu.sync_copy(outv.at[pl.ds(0, npw)], out_hbm.at[pl.ds(base, npw)])
    pltpu.sync_copy(outv.at[pl.ds(npw, npw)], out_hbm.at[pl.ds(n + base, npw)])
    pltpu.sync_copy(outv.at[pl.ds(2 * npw, npw)],
                    out_hbm.at[pl.ds(2 * n + base, npw)])


def _apply_sc(pred_t, pos_t, table, seg):
    n = pred_t.shape[1]
    npw = n // (SC_NC * SC_NS)
    mesh = plsc.VectorSubcoreMesh(core_axis_name="c", subcore_axis_name="s")
    run = pl.kernel(
        _apply_sc_body,
        mesh=mesh,
        compiler_params=pltpu.CompilerParams(
            needs_layout_passes=False, use_tc_tiling_on_sc=False,
            skip_device_barrier=True),
        out_type=jax.ShapeDtypeStruct((3 * n,), jnp.float32),
        scratch_types=[
            pltpu.VMEM((npw,), jnp.float32),
            pltpu.VMEM((npw,), jnp.float32),
            pltpu.VMEM((npw,), jnp.float32),
            pltpu.VMEM((npw,), jnp.float32),
            pltpu.VMEM((npw,), jnp.float32),
            pltpu.VMEM((npw,), jnp.float32),
            pltpu.VMEM((npw,), jnp.int32),
            pltpu.VMEM((16 * 512,), jnp.float32),
            pltpu.VMEM((3 * npw,), jnp.float32),
        ],
    )
    return run(pred_t.reshape(3 * n), pos_t.reshape(3 * n),
               table.reshape(-1), seg).reshape(3, n).T


def _apply_body(pred_ref, pos_ref, sw_ref, ew_ref, bj_ref, table_ref, out_ref):
    t = pl.program_id(0)
    bj = bj_ref[0]
    giota = jax.lax.broadcasted_iota(jnp.int32, (512, 1), 0)
    eq = jnp.where(giota == bj, 1.0, 0.0)
    twin = jnp.dot(table_ref[...], eq, preferred_element_type=jnp.float32)

    ids = jax.lax.broadcasted_iota(jnp.int32, (1, N_TILE), 1) + t * N_TILE
    sw = sw_ref[0]
    ew = ew_ref[0]
    onehot = jnp.where((ids >= sw) & (ids < ew), 1.0, 0.0)
    vals = jnp.dot(twin, onehot, preferred_element_type=jnp.float32)

    pred = pred_ref[...]
    pos = pos_ref[...]
    rx = pos[0:1] - vals[3:4]
    ry = pos[1:2] - vals[4:5]
    rz = pos[2:3] - vals[5:6]
    dx, dy, dz = _cross_rows(rx, ry, rz, vals[6:7], vals[7:8], vals[8:9])
    ox = pred[0:1] - vals[0:1] + dx
    oy = pred[1:2] - vals[1:2] + dy
    oz = pred[2:3] - vals[2:3] + dz
    out_ref[...] = jnp.concatenate([ox, oy, oz], axis=0)


def kernel(x, positions, cell, n_node, W1, b1, W2, b2):
    N = x.shape[0]
    B = n_node.shape[0]
    T = N // N_TILE

    nn = n_node.astype(jnp.int32)
    ends = jnp.cumsum(nn)
    starts = ends - nn
    tile_starts = jnp.arange(T, dtype=jnp.int32) * N_TILE
    base = jnp.searchsorted(ends, tile_starts, side='right').astype(jnp.int32)
    win = base[:, None] + jnp.arange(WIN, dtype=jnp.int32)[None, :]
    valid = win < B
    winc = jnp.clip(win, 0, B - 1)
    s_w = jnp.where(valid, starts[winc], N).astype(jnp.int32)
    e_w = jnp.where(valid, ends[winc], N).astype(jnp.int32)
    bj = jnp.where(valid, win, -1).astype(jnp.int32)
    sw3 = s_w.reshape(T, WIN, 1)
    ew3 = e_w.reshape(T, WIN, 1)
    bjc = bj.reshape(T, WIN, 1)
    bjr = bj.reshape(T, 1, WIN)
    nnf = n_node.astype(jnp.float32).reshape(1, B)
    cell_t = cell.reshape(B, 9).T
    pos_t = positions.T

    pred_t, table, seg = pl.pallas_call(
        _mlp_moments_body,
        grid=(T,),
        in_specs=[
            pl.BlockSpec((N_TILE, 128), lambda t: (t, 0)),
            pl.BlockSpec((128, 128), lambda t: (0, 0)),
            pl.BlockSpec((1, 128), lambda t: (0, 0)),
            pl.BlockSpec((128, 3), lambda t: (0, 0)),
            pl.BlockSpec((3, 1), lambda t: (0, 0)),
            pl.BlockSpec((3, N_TILE), lambda t: (0, t)),
            pl.BlockSpec((1, WIN, 1), lambda t: (t, 0, 0)),
            pl.BlockSpec((1, WIN, 1), lambda t: (t, 0, 0)),
            pl.BlockSpec((1, WIN, 1), lambda t: (t, 0, 0)),
            pl.BlockSpec((1, B), lambda t: (0, 0)),
            pl.BlockSpec((9, B), lambda t: (0, 0)),
        ],
        out_specs=[
            pl.BlockSpec((3, N_TILE), lambda t: (0, t)),
            pl.BlockSpec((16, B), lambda t: (0, 0)),
            pl.BlockSpec((1, N_TILE), lambda t: (0, t)),
        ],
        out_shape=[
            jax.ShapeDtypeStruct((3, N), jnp.float32),
            jax.ShapeDtypeStruct((16, B), jnp.float32),
            jax.ShapeDtypeStruct((1, N), jnp.int32),
        ],
        scratch_shapes=[pltpu.VMEM((16, B), jnp.float32)],
        compiler_params=pltpu.CompilerParams(
            dimension_semantics=("arbitrary",)),
    )(x, W1, b1.reshape(1, 128), W2, b2.reshape(3, 1), pos_t, sw3, ew3, bjc,
      nnf, cell_t)

    return _apply_sc(pred_t, pos_t, table, seg.reshape(N))


# speedup vs baseline: 87.6856x; 1.2839x over previous
import jax
import jax.numpy as jnp
from jax import lax
from jax.experimental import pallas as pl
from jax.experimental.pallas import tpu as pltpu
from jax.experimental.pallas import tpu_sc as plsc

N_TILE = 4096
WIN = 32


def _cross_rows(ax, ay, az, bx, by, bz):
    return (ay * bz - az * by, az * bx - ax * bz, ax * by - ay * bx)


def _solve_from_moments(mom, nn, cell):
    ninv = 1.0 / nn
    ax_, ay_, az_ = mom[0:1], mom[1:2], mom[2:3]
    px_, py_, pz_ = mom[3:4], mom[4:5], mom[5:6]
    cx_, cy_, cz_ = mom[6:7], mom[7:8], mom[8:9]
    q = mom[9:10]
    oxx, oyy, ozz = mom[10:11], mom[11:12], mom[12:13]
    oxy, oxz, oyz = mom[13:14], mom[14:15], mom[15:16]

    mean_x, mean_y, mean_z = ax_ * ninv, ay_ * ninv, az_ * ninv
    com_x, com_y, com_z = px_ * ninv, py_ * ninv, pz_ * ninv
    kx, ky, kz = _cross_rows(px_, py_, pz_, ax_, ay_, az_)
    tx = cx_ - kx * ninv
    ty = cy_ - ky * ninv
    tz = cz_ - kz * ninv
    s = q - (px_ * px_ + py_ * py_ + pz_ * pz_) * ninv
    a = oxx - px_ * px_ * ninv - s
    d = oyy - py_ * py_ * ninv - s
    f = ozz - pz_ * pz_ * ninv - s
    b = oxy - px_ * py_ * ninv
    c = oxz - px_ * pz_ * ninv
    e = oyz - py_ * pz_ * ninv

    det = a * (d * f - e * e) - b * (b * f - e * c) + c * (b * e - d * c)
    dinv = 1.0 / det
    i00 = d * f - e * e
    i01 = c * e - b * f
    i02 = b * e - c * d
    i11 = a * f - c * c
    i12 = b * c - a * e
    i22 = a * d - b * b
    mux = -(i00 * tx + i01 * ty + i02 * tz) * dinv
    muy = -(i01 * tx + i11 * ty + i12 * tz) * dinv
    muz = -(i02 * tx + i12 * ty + i22 * tz) * dinv
    rx = tx + a * mux + b * muy + c * muz
    ry = ty + b * mux + d * muy + e * muz
    rz = tz + c * mux + e * muy + f * muz
    mux = mux - (i00 * rx + i01 * ry + i02 * rz) * dinv
    muy = muy - (i01 * rx + i11 * ry + i12 * rz) * dinv
    muz = muz - (i02 * rx + i12 * ry + i22 * rz) * dinv

    nopbc = jnp.all(cell == 0.0, axis=0, keepdims=True)
    zero = jnp.zeros_like(mux)
    mux = jnp.where(nopbc, mux, zero)
    muy = jnp.where(nopbc, muy, zero)
    muz = jnp.where(nopbc, muz, zero)

    return jnp.concatenate(
        [mean_x, mean_y, mean_z, com_x, com_y, com_z, mux, muy, muz,
         zero, zero, zero, zero, zero, zero, zero], axis=0)


def _mlp_moments_body(x_ref, w1_ref, b1_ref, w2_ref, b2_ref, pos_ref,
                      sw_ref, ew_ref, bj_ref, nn_ref, cell_ref,
                      pred_ref, table_ref, seg_ref, mom_ref):
    t = pl.program_id(0)
    nt = pl.num_programs(0)
    h = jax.nn.gelu(jnp.dot(x_ref[...], w1_ref[...],
                            preferred_element_type=jnp.float32) + b1_ref[...])
    pred = jax.lax.dot_general(w2_ref[...], h, (((0,), (1,)), ((), ())),
                               preferred_element_type=jnp.float32) + b2_ref[...]
    pred_ref[...] = pred

    pos = pos_ref[...]
    px, py, pz = pos[0:1], pos[1:2], pos[2:3]
    fx, fy, fz = pred[0:1], pred[1:2], pred[2:3]
    cx, cy, cz = _cross_rows(px, py, pz, fx, fy, fz)
    rsq = px * px + py * py + pz * pz
    feats = jnp.concatenate(
        [fx, fy, fz, px, py, pz, cx, cy, cz, rsq,
         px * px, py * py, pz * pz, px * py, px * pz, py * pz], axis=0)

    ids = jax.lax.broadcasted_iota(jnp.int32, (1, N_TILE), 1) + t * N_TILE
    sw = sw_ref[0]
    ew = ew_ref[0]
    onehot = jnp.where((ids >= sw) & (ids < ew), 1.0, 0.0)
    part = jax.lax.dot_general(feats, onehot, (((1,), (1,)), ((), ())),
                               preferred_element_type=jnp.float32)

    bj = bj_ref[0]
    giota = jax.lax.broadcasted_iota(jnp.int32, (WIN, mom_ref.shape[1]), 1)
    eqw = jnp.where(giota == bj, 1.0, 0.0)
    contrib = jnp.dot(part, eqw, preferred_element_type=jnp.float32)

    inwin = (ids >= sw) & (ids < ew)
    seg_ref[...] = jnp.sum(jnp.where(inwin, bj, 0), axis=0, keepdims=True)

    @pl.when(t == 0)
    def _():
        mom_ref[...] = contrib

    @pl.when(t > 0)
    def _():
        mom_ref[...] += contrib

    @pl.when(t == nt - 1)
    def _():
        table_ref[...] = _solve_from_moments(mom_ref[...], nn_ref[...],
                                             cell_ref[...])


SC_NC = 2
SC_NS = 16
SC_L = 16


def _apply_sc_body(pred_hbm, pos_hbm, tab_hbm, seg_hbm, out_hbm,
                   fxv, fyv, fzv, pxv, pyv, pzv, segv, tabv, outv):
    npw = segv.shape[0]
    n = pred_hbm.shape[0] // 3
    wid = lax.axis_index("s") * SC_NC + lax.axis_index("c")
    base = wid * npw
    pltpu.sync_copy(pred_hbm.at[pl.ds(base, npw)], fxv)
    pltpu.sync_copy(pred_hbm.at[pl.ds(n + base, npw)], fyv)
    pltpu.sync_copy(pred_hbm.at[pl.ds(2 * n + base, npw)], fzv)
    pltpu.sync_copy(pos_hbm.at[pl.ds(base, npw)], pxv)
    pltpu.sync_copy(pos_hbm.at[pl.ds(n + base, npw)], pyv)
    pltpu.sync_copy(pos_hbm.at[pl.ds(2 * n + base, npw)], pzv)
    pltpu.sync_copy(seg_hbm.at[pl.ds(base, npw)], segv)
    pltpu.sync_copy(tab_hbm, tabv)

    nb = tab_hbm.shape[0] // 16

    @plsc.parallel_loop(0, npw // SC_L, unroll=8)
    def chunk(k):
        off = k * SC_L
        idx = segv[pl.ds(off, SC_L)]
        def gat(comp):
            return plsc.load_gather(tabv, [idx + comp * nb])
        mx, my, mz = gat(0), gat(1), gat(2)
        ox_, oy_, oz_ = gat(3), gat(4), gat(5)
        ux, uy, uz = gat(6), gat(7), gat(8)
        rx = pxv[pl.ds(off, SC_L)] - ox_
        ry = pyv[pl.ds(off, SC_L)] - oy_
        rz = pzv[pl.ds(off, SC_L)] - oz_
        outv[pl.ds(off, SC_L)] = fxv[pl.ds(off, SC_L)] - mx + (ry * uz - rz * uy)
        outv[pl.ds(npw + off, SC_L)] = fyv[pl.ds(off, SC_L)] - my + (rz * ux - rx * uz)
        outv[pl.ds(2 * npw + off, SC_L)] = fzv[pl.ds(off, SC_L)] - mz + (rx * uy - ry * ux)

    pltpu.sync_copy(outv.at[pl.ds(0, npw)], out_hbm.at[pl.ds(base, npw)])
    pltpu.sync_copy(outv.at[pl.ds(npw, npw)], out_hbm.at[pl.ds(n + base, npw)])
    pltpu.sync_copy(outv.at[pl.ds(2 * npw, npw)],
                    out_hbm.at[pl.ds(2 * n + base, npw)])


def _apply_sc(pred_t, pos_t, table, seg):
    n = pred_t.shape[1]
    npw = n // (SC_NC * SC_NS)
    mesh = plsc.VectorSubcoreMesh(core_axis_name="c", subcore_axis_name="s")
    run = pl.kernel(
        _apply_sc_body,
        mesh=mesh,
        compiler_params=pltpu.CompilerParams(
            needs_layout_passes=False, use_tc_tiling_on_sc=False,
            skip_device_barrier=True),
        out_type=jax.ShapeDtypeStruct((3 * n,), jnp.float32),
        scratch_types=[
            pltpu.VMEM((npw,), jnp.float32),
            pltpu.VMEM((npw,), jnp.float32),
            pltpu.VMEM((npw,), jnp.float32),
            pltpu.VMEM((npw,), jnp.float32),
            pltpu.VMEM((npw,), jnp.float32),
            pltpu.VMEM((npw,), jnp.float32),
            pltpu.VMEM((npw,), jnp.int32),
            pltpu.VMEM((16 * 512,), jnp.float32),
            pltpu.VMEM((3 * npw,), jnp.float32),
        ],
    )
    return run(pred_t.reshape(3 * n), pos_t.reshape(3 * n),
               table.reshape(-1), seg).reshape(3, n).T


def _apply_body(pred_ref, pos_ref, sw_ref, ew_ref, bj_ref, table_ref, out_ref):
    t = pl.program_id(0)
    bj = bj_ref[0]
    giota = jax.lax.broadcasted_iota(jnp.int32, (512, 1), 0)
    eq = jnp.where(giota == bj, 1.0, 0.0)
    twin = jnp.dot(table_ref[...], eq, preferred_element_type=jnp.float32)

    ids = jax.lax.broadcasted_iota(jnp.int32, (1, N_TILE), 1) + t * N_TILE
    sw = sw_ref[0]
    ew = ew_ref[0]
    onehot = jnp.where((ids >= sw) & (ids < ew), 1.0, 0.0)
    vals = jnp.dot(twin, onehot, preferred_element_type=jnp.float32)

    pred = pred_ref[...]
    pos = pos_ref[...]
    rx = pos[0:1] - vals[3:4]
    ry = pos[1:2] - vals[4:5]
    rz = pos[2:3] - vals[5:6]
    dx, dy, dz = _cross_rows(rx, ry, rz, vals[6:7], vals[7:8], vals[8:9])
    ox = pred[0:1] - vals[0:1] + dx
    oy = pred[1:2] - vals[1:2] + dy
    oz = pred[2:3] - vals[2:3] + dz
    out_ref[...] = jnp.concatenate([ox, oy, oz], axis=0)


def kernel(x, positions, cell, n_node, W1, b1, W2, b2):
    N = x.shape[0]
    B = n_node.shape[0]
    T = N // N_TILE

    nn = n_node.astype(jnp.int32)
    ends = jnp.cumsum(nn)
    starts = ends - nn
    tile_starts = jnp.arange(T, dtype=jnp.int32) * N_TILE
    base = jnp.searchsorted(ends, tile_starts, side='right').astype(jnp.int32)
    win = base[:, None] + jnp.arange(WIN, dtype=jnp.int32)[None, :]
    valid = win < B
    winc = jnp.clip(win, 0, B - 1)
    s_w = jnp.where(valid, starts[winc], N).astype(jnp.int32)
    e_w = jnp.where(valid, ends[winc], N).astype(jnp.int32)
    bj = jnp.where(valid, win, -1).astype(jnp.int32)
    sw3 = s_w.reshape(T, WIN, 1)
    ew3 = e_w.reshape(T, WIN, 1)
    bjc = bj.reshape(T, WIN, 1)
    bjr = bj.reshape(T, 1, WIN)
    nnf = n_node.astype(jnp.float32).reshape(1, B)
    cell_t = cell.reshape(B, 9).T
    pos_t = positions.T

    pred_t, table, seg = pl.pallas_call(
        _mlp_moments_body,
        grid=(T,),
        in_specs=[
            pl.BlockSpec((N_TILE, 128), lambda t: (t, 0)),
            pl.BlockSpec((128, 128), lambda t: (0, 0)),
            pl.BlockSpec((1, 128), lambda t: (0, 0)),
            pl.BlockSpec((128, 3), lambda t: (0, 0)),
            pl.BlockSpec((3, 1), lambda t: (0, 0)),
            pl.BlockSpec((3, N_TILE), lambda t: (0, t)),
            pl.BlockSpec((1, WIN, 1), lambda t: (t, 0, 0)),
            pl.BlockSpec((1, WIN, 1), lambda t: (t, 0, 0)),
            pl.BlockSpec((1, WIN, 1), lambda t: (t, 0, 0)),
            pl.BlockSpec((1, B), lambda t: (0, 0)),
            pl.BlockSpec((9, B), lambda t: (0, 0)),
        ],
        out_specs=[
            pl.BlockSpec((3, N_TILE), lambda t: (0, t)),
            pl.BlockSpec((16, B), lambda t: (0, 0)),
            pl.BlockSpec((1, N_TILE), lambda t: (0, t)),
        ],
        out_shape=[
            jax.ShapeDtypeStruct((3, N), jnp.float32),
            jax.ShapeDtypeStruct((16, B), jnp.float32),
            jax.ShapeDtypeStruct((1, N), jnp.int32),
        ],
        scratch_shapes=[pltpu.VMEM((16, B), jnp.float32)],
        compiler_params=pltpu.CompilerParams(
            dimension_semantics=("arbitrary",)),
    )(x, W1, b1.reshape(1, 128), W2, b2.reshape(3, 1), pos_t, sw3, ew3, bjc,
      nnf, cell_t)

    return _apply_sc(pred_t, pos_t, table, seg.reshape(N))
